# baseline (speedup 1.0000x reference)
import functools

import jax
import jax.numpy as jnp
from jax import lax
from jax.experimental import pallas as pl
from jax.experimental.pallas import tpu as pltpu
from jax.experimental.pallas import tpu_sc as plsc

_B = 1024
_A = 100
_ACT = 1000
_NC, _NS = 2, 16
_NW = _NC * _NS
_RPW = _B // _NW
_AP = 112
_NG = _AP // 16
_CNT = 1024


def _sc_body(a_hbm, c1_hbm, c2_hbm, tau_hbm, out_hbm,
             a_v, o_v, cnt0_v, cnt1_v, d_v, c2_v, tau_v):
    w = lax.axis_index("s") * _NC + lax.axis_index("c")
    base = w * (_RPW * _AP)
    pltpu.sync_copy(a_hbm.at[pl.ds(base, _RPW * _AP)], a_v)
    pltpu.sync_copy(c1_hbm, d_v)
    pltpu.sync_copy(c2_hbm, c2_v)

    zero16 = jnp.zeros((16,), jnp.float32)
    one16 = jnp.ones((16,), jnp.float32)
    for i in range(_CNT // 16):
        cnt0_v[pl.ds(16 * i, 16)] = zero16
        cnt1_v[pl.ds(16 * i, 16)] = zero16

    pltpu.sync_copy(tau_hbm, tau_v)
    for i in range(_CNT // 16):
        s = pl.ds(16 * i, 16)
        d_v[s] = tau_v[s] - d_v[s]

    for r in range(_RPW):
        cnt_v = cnt0_v if r % 2 == 0 else cnt1_v
        ab = r * _AP
        idx = [a_v[pl.ds(ab + 16 * g, 16)] for g in range(_NG)]
        for g in range(_NG):
            plsc.addupdate_scatter(cnt_v, [idx[g]], one16)
        for g in range(_NG):
            ld = plsc.load_gather(cnt_v, [idx[g]])
            dg = plsc.load_gather(d_v, [idx[g]])
            c2g = plsc.load_gather(c2_v, [idx[g]])
            o_v[pl.ds(ab + 16 * g, 16)] = ld * (dg - c2g * ld)
        for g in range(_NG):
            plsc.store_scatter(cnt_v, [idx[g]], zero16)
    pltpu.sync_copy(o_v, out_hbm.at[pl.ds(base, _RPW * _AP)])


@jax.jit
def kernel(a_joint, c1, c2, tau):
    a32 = a_joint.astype(jnp.int32)
    pad = jnp.broadcast_to(
        jnp.arange(_ACT, _ACT + (_AP - _A), dtype=jnp.int32), (_B, _AP - _A))
    a_pad = jnp.concatenate([a32, pad], axis=1).reshape(-1)
    c1p = jnp.pad(c1, (0, _CNT - _ACT))
    c2p = jnp.pad(c2, (0, _CNT - _ACT))
    taup = jnp.pad(tau, (0, _CNT - _ACT))

    mesh = plsc.VectorSubcoreMesh(
        core_axis_name="c", subcore_axis_name="s",
        num_cores=_NC, num_subcores=_NS)
    out = pl.kernel(
        _sc_body,
        out_type=jax.ShapeDtypeStruct((_B * _AP,), jnp.float32),
        mesh=mesh,
        compiler_params=pltpu.CompilerParams(needs_layout_passes=False),
        scratch_types=[
            pltpu.VMEM((_RPW * _AP,), jnp.int32),
            pltpu.VMEM((_RPW * _AP,), jnp.float32),
            pltpu.VMEM((_CNT,), jnp.float32),
            pltpu.VMEM((_CNT,), jnp.float32),
            pltpu.VMEM((_CNT,), jnp.float32),
            pltpu.VMEM((_CNT,), jnp.float32),
            pltpu.VMEM((_CNT,), jnp.float32),
        ],
    )(a_pad, c1p, c2p, taup)
    return out.reshape(_B, _AP)[:, :_A]


# speedup vs baseline: 1.1388x; 1.1388x over previous
import functools

import jax
import jax.numpy as jnp
from jax import lax
from jax.experimental import pallas as pl
from jax.experimental.pallas import tpu as pltpu
from jax.experimental.pallas import tpu_sc as plsc

_B = 1024
_A = 100
_ACT = 1000
_NC, _NS = 2, 16
_NW = _NC * _NS
_RPW = _B // _NW
_SR = 4
_NSR = _RPW // _SR
_SRW = _SR * _A
_NV = _SRW // 16
_CHUNK = _RPW * _A
_PRM = 1008
_CNT = 1024


def _sc_body(a_hbm, c1_hbm, c2_hbm, tau_hbm, out_hbm,
             a_v, o_v, d_v, c2_v, tau_v, cb0, cb1, cb2, cb3):
    w = lax.axis_index("s") * _NC + lax.axis_index("c")
    base = w * _CHUNK
    pltpu.sync_copy(a_hbm.at[pl.ds(base, _CHUNK)], a_v)
    pltpu.sync_copy(c1_hbm, d_v.at[pl.ds(0, _ACT)])
    pltpu.sync_copy(c2_hbm, c2_v.at[pl.ds(0, _ACT)])
    pltpu.sync_copy(tau_hbm, tau_v.at[pl.ds(0, _ACT)])

    zero16 = jnp.zeros((16,), jnp.float32)
    one16 = jnp.ones((16,), jnp.float32)
    cbs = [cb0, cb1, cb2, cb3]
    for i in range(_CNT // 16):
        for cb in cbs:
            cb[pl.ds(16 * i, 16)] = zero16
    for i in range(_PRM // 16):
        s = pl.ds(16 * i, 16)
        d_v[s] = tau_v[s] - d_v[s]

    lane = lax.broadcasted_iota(jnp.int32, (16,), 0)
    def split(k):
        lo_row = (16 * k) // _A
        hi_row = (16 * k + 15) // _A
        cut = _A * hi_row - 16 * k
        return lo_row, hi_row, cut

    def superrow(s, carry):
        sb = s * _SRW
        idx = [a_v[pl.ds(sb + 16 * k, 16)] for k in range(_NV)]
        for k in range(_NV):
            lo, hi, cut = split(k)
            if lo == hi:
                plsc.addupdate_scatter(cbs[lo], [idx[k]], one16)
            else:
                m = lane < cut
                plsc.addupdate_scatter(cbs[lo], [idx[k]], one16, mask=m)
                plsc.addupdate_scatter(cbs[hi], [idx[k]], one16,
                                       mask=jnp.logical_not(m))
        for k in range(_NV):
            lo, hi, cut = split(k)
            if lo == hi:
                ld = plsc.load_gather(cbs[lo], [idx[k]])
            else:
                glo = plsc.load_gather(cbs[lo], [idx[k]])
                ghi = plsc.load_gather(cbs[hi], [idx[k]])
                ld = jnp.where(lane < cut, glo, ghi)
            dg = plsc.load_gather(d_v, [idx[k]])
            c2g = plsc.load_gather(c2_v, [idx[k]])
            o_v[pl.ds(sb + 16 * k, 16)] = ld * (dg - c2g * ld)
        for k in range(_NV):
            lo, hi, cut = split(k)
            if lo == hi:
                plsc.store_scatter(cbs[lo], [idx[k]], zero16)
            else:
                m = lane < cut
                plsc.store_scatter(cbs[lo], [idx[k]], zero16, mask=m)
                plsc.store_scatter(cbs[hi], [idx[k]], zero16,
                                   mask=jnp.logical_not(m))
        return carry

    lax.fori_loop(0, _NSR, superrow, 0)
    pltpu.sync_copy(o_v, out_hbm.at[pl.ds(base, _CHUNK)])


@jax.jit
def kernel(a_joint, c1, c2, tau):
    a_flat = a_joint.astype(jnp.int32).reshape(-1)
    mesh = plsc.VectorSubcoreMesh(
        core_axis_name="c", subcore_axis_name="s",
        num_cores=_NC, num_subcores=_NS)
    out = pl.kernel(
        _sc_body,
        out_type=jax.ShapeDtypeStruct((_B * _A,), jnp.float32),
        mesh=mesh,
        compiler_params=pltpu.CompilerParams(needs_layout_passes=False),
        scratch_types=[
            pltpu.VMEM((_CHUNK,), jnp.int32),
            pltpu.VMEM((_CHUNK,), jnp.float32),
            pltpu.VMEM((_PRM,), jnp.float32),
            pltpu.VMEM((_PRM,), jnp.float32),
            pltpu.VMEM((_PRM,), jnp.float32),
            pltpu.VMEM((_CNT,), jnp.float32),
            pltpu.VMEM((_CNT,), jnp.float32),
            pltpu.VMEM((_CNT,), jnp.float32),
            pltpu.VMEM((_CNT,), jnp.float32),
        ],
    )(a_flat, c1, c2, tau)
    return out.reshape(_B, _A)


# speedup vs baseline: 1.1455x; 1.0059x over previous
import functools

import jax
import jax.numpy as jnp
from jax import lax
from jax.experimental import pallas as pl
from jax.experimental.pallas import tpu as pltpu
from jax.experimental.pallas import tpu_sc as plsc

_B = 1024
_A = 100
_ACT = 1000
_NC, _NS = 2, 16
_NW = _NC * _NS
_RPW = _B // _NW
_SR = 4
_NSR = _RPW // _SR
_SRW = _SR * _A
_NV = _SRW // 16
_CHUNK = _RPW * _A
_PRM = 1008
_CNT = 1024


def _sc_body(a_hbm, c1_hbm, c2_hbm, tau_hbm, out_hbm,
             a_v, o_v, d_v, c2_v, tau_v, cb0, cb1, cb2, cb3):
    w = lax.axis_index("s") * _NC + lax.axis_index("c")
    base = w * _CHUNK
    pltpu.sync_copy(a_hbm.at[pl.ds(base, _CHUNK)], a_v)
    pltpu.sync_copy(c1_hbm, d_v.at[pl.ds(0, _ACT)])
    pltpu.sync_copy(c2_hbm, c2_v.at[pl.ds(0, _ACT)])
    pltpu.sync_copy(tau_hbm, tau_v.at[pl.ds(0, _ACT)])

    zero16 = jnp.zeros((16,), jnp.float32)
    one16 = jnp.ones((16,), jnp.float32)
    cbs = [cb0, cb1, cb2, cb3]
    for i in range(_CNT // 16):
        for cb in cbs:
            cb[pl.ds(16 * i, 16)] = zero16
    for i in range(_PRM // 16):
        s = pl.ds(16 * i, 16)
        d_v[s] = tau_v[s] - d_v[s]

    lane = lax.broadcasted_iota(jnp.int32, (16,), 0)
    tail_mask = lane >= 12

    def row(r, carry):
        rb = r * _A
        offs = [16 * g for g in range(6)] + [_A - 16]
        cnt = cbs[0]
        idx = [a_v[pl.ds(rb + o, 16)] for o in offs]
        for g in range(6):
            plsc.addupdate_scatter(cnt, [idx[g]], one16)
        plsc.addupdate_scatter(cnt, [idx[6]], one16, mask=tail_mask)
        for g in range(7):
            ld = plsc.load_gather(cnt, [idx[g]])
            dg = plsc.load_gather(d_v, [idx[g]])
            c2g = plsc.load_gather(c2_v, [idx[g]])
            o_v[pl.ds(rb + offs[g], 16)] = ld * (dg - c2g * ld)
        for g in range(6):
            plsc.store_scatter(cnt, [idx[g]], zero16)
        plsc.store_scatter(cnt, [idx[6]], zero16, mask=tail_mask)
        return carry

    lax.fori_loop(0, _RPW, row, 0)
    pltpu.sync_copy(o_v, out_hbm.at[pl.ds(base, _CHUNK)])


@jax.jit
def kernel(a_joint, c1, c2, tau):
    a_flat = a_joint.astype(jnp.int32).reshape(-1)
    mesh = plsc.VectorSubcoreMesh(
        core_axis_name="c", subcore_axis_name="s",
        num_cores=_NC, num_subcores=_NS)
    out = pl.kernel(
        _sc_body,
        out_type=jax.ShapeDtypeStruct((_B * _A,), jnp.float32),
        mesh=mesh,
        compiler_params=pltpu.CompilerParams(needs_layout_passes=False),
        scratch_types=[
            pltpu.VMEM((_CHUNK,), jnp.int32),
            pltpu.VMEM((_CHUNK,), jnp.float32),
            pltpu.VMEM((_PRM,), jnp.float32),
            pltpu.VMEM((_PRM,), jnp.float32),
            pltpu.VMEM((_PRM,), jnp.float32),
            pltpu.VMEM((_CNT,), jnp.float32),
            pltpu.VMEM((_CNT,), jnp.float32),
            pltpu.VMEM((_CNT,), jnp.float32),
            pltpu.VMEM((_CNT,), jnp.float32),
        ],
    )(a_flat, c1, c2, tau)
    return out.reshape(_B, _A)


# speedup vs baseline: 1.1778x; 1.0281x over previous
import functools

import jax
import jax.numpy as jnp
from jax import lax
from jax.experimental import pallas as pl
from jax.experimental.pallas import tpu as pltpu
from jax.experimental.pallas import tpu_sc as plsc

_B = 1024
_A = 100
_ACT = 1000
_NC, _NS = 2, 16
_NW = _NC * _NS
_RPW = _B // _NW
_SR = 4
_NSR = _RPW // _SR
_SRW = _SR * _A
_NV = _SRW // 16
_CHUNK = _RPW * _A
_PRM = 1008
_CNT = 1024


def _sc_body(a_hbm, c1_hbm, c2_hbm, tau_hbm, out_hbm,
             a_v, o_v, d_v, c2_v, tau_v, cb0, cb1, cb2, cb3):
    w = lax.axis_index("s") * _NC + lax.axis_index("c")
    rbase = w * _RPW
    pltpu.sync_copy(a_hbm.at[pl.ds(rbase, _RPW)], a_v)
    pltpu.sync_copy(c1_hbm, d_v.at[pl.ds(0, _ACT)])
    pltpu.sync_copy(c2_hbm, c2_v.at[pl.ds(0, _ACT)])
    pltpu.sync_copy(tau_hbm, tau_v.at[pl.ds(0, _ACT)])

    zero16 = jnp.zeros((16,), jnp.float32)
    one16 = jnp.ones((16,), jnp.float32)
    cbs = [cb0, cb1, cb2, cb3]
    for i in range(_CNT // 16):
        for cb in cbs:
            cb[pl.ds(16 * i, 16)] = zero16
    for i in range(_PRM // 16):
        s = pl.ds(16 * i, 16)
        d_v[s] = tau_v[s] - d_v[s]

    lane = lax.broadcasted_iota(jnp.int32, (16,), 0)
    tail_mask = lane >= 12

    def row(r, carry):
        offs = [16 * g for g in range(6)] + [_A - 16]
        cnt = cbs[0]
        idx = [a_v[r, pl.ds(o, 16)] for o in offs]
        for g in range(6):
            plsc.addupdate_scatter(cnt, [idx[g]], one16)
        plsc.addupdate_scatter(cnt, [idx[6]], one16, mask=tail_mask)
        for g in range(7):
            ld = plsc.load_gather(cnt, [idx[g]])
            dg = plsc.load_gather(d_v, [idx[g]])
            c2g = plsc.load_gather(c2_v, [idx[g]])
            o_v[r, pl.ds(offs[g], 16)] = ld * (dg - c2g * ld)
        for g in range(6):
            plsc.store_scatter(cnt, [idx[g]], zero16)
        plsc.store_scatter(cnt, [idx[6]], zero16, mask=tail_mask)
        return carry

    lax.fori_loop(0, _RPW, row, 0)
    pltpu.sync_copy(o_v, out_hbm.at[pl.ds(rbase, _RPW)])


@jax.jit
def kernel(a_joint, c1, c2, tau):
    a32 = a_joint.astype(jnp.int32)
    mesh = plsc.VectorSubcoreMesh(
        core_axis_name="c", subcore_axis_name="s",
        num_cores=_NC, num_subcores=_NS)
    return pl.kernel(
        _sc_body,
        out_type=jax.ShapeDtypeStruct((_B, _A), jnp.float32),
        mesh=mesh,
        compiler_params=pltpu.CompilerParams(needs_layout_passes=False),
        scratch_types=[
            pltpu.VMEM((_RPW, _A), jnp.int32),
            pltpu.VMEM((_RPW, _A), jnp.float32),
            pltpu.VMEM((_PRM,), jnp.float32),
            pltpu.VMEM((_PRM,), jnp.float32),
            pltpu.VMEM((_PRM,), jnp.float32),
            pltpu.VMEM((_CNT,), jnp.float32),
            pltpu.VMEM((_CNT,), jnp.float32),
            pltpu.VMEM((_CNT,), jnp.float32),
            pltpu.VMEM((_CNT,), jnp.float32),
        ],
    )(a32, c1, c2, tau)
